# baseline (speedup 1.0000x reference)
import functools

import jax
import jax.numpy as jnp
from jax import lax
from jax.experimental import pallas as pl
from jax.experimental.pallas import tpu as pltpu
from jax.experimental.pallas import tpu_sc as plsc

KNBRS = 9
NGROUPS = 4
ROWBLK = 784
NWORKERS = 32
SUBCHUNK = 56


def _knn_body(xt_ref, xtt_ref, idx_ref):
    xb = xt_ref[0]
    xt_all = xtt_ref[0]
    R = xb.shape[0]
    N = xt_all.shape[1]
    rn = jnp.sqrt(jnp.sum(xb * xb, axis=1, keepdims=True))
    xbn = xb / jnp.maximum(rn, 1e-12)
    cn = jnp.sqrt(jnp.sum(xt_all * xt_all, axis=0, keepdims=True))
    xcn = xt_all / jnp.maximum(cn, 1e-12)
    inner = jnp.dot(xbn, xcn, preferred_element_type=jnp.float32)
    sq_row = jnp.sum(xbn * xbn, axis=1, keepdims=True)
    sq_col = jnp.sum(xcn * xcn, axis=0, keepdims=True)
    neg = 2.0 * inner - sq_row - sq_col
    iota = lax.broadcasted_iota(jnp.int32, (R, N), 1)
    masked_out = jnp.float32(-3.0e38)
    picks = []
    for _ in range(KNBRS):
        a = jnp.argmax(neg, axis=1).astype(jnp.int32)[:, None]
        picks.append(a)
        neg = jnp.where(iota == a, masked_out, neg)
    picks.append(jnp.zeros((R, 16 - len(picks)), jnp.int32))
    idx_ref[0] = jnp.concatenate(picks, axis=1)


def _conv_body(xt_ref, xg_ref, wx_ref, wj_ref, out_ref):
    nt = (((1,), (1,)), ((), ()))
    acc = jax.lax.dot_general(wx_ref[...], xt_ref[0], nt,
                              preferred_element_type=jnp.float32)
    acc += jax.lax.dot_general(wj_ref[...], xg_ref[0], nt,
                               preferred_element_type=jnp.float32)
    out_ref[0] = jnp.maximum(acc, 0.0)


def _gather_body(xt_ref, gidx_ref, out_ref, idx_v, gbuf, obuf, sem):
    n_total = xt_ref.shape[0]
    cdim = xt_ref.shape[1]
    per_w = n_total // NWORKERS
    wid = lax.axis_index("s") * 2 + lax.axis_index("c")
    base = wid * per_w
    pltpu.sync_copy(gidx_ref.at[pl.ds(wid * (KNBRS * per_w), KNBRS * per_w)],
                    idx_v)
    nsub = per_w // SUBCHUNK
    for j in range(nsub):
        cps = [
            pltpu.async_copy(
                xt_ref.at[idx_v.at[pl.ds(k * per_w + j * SUBCHUNK, SUBCHUNK)]],
                gbuf.at[k], sem)
            for k in range(KNBRS)
        ]
        for cp in cps:
            cp.wait()

        def row_max(r, _):
            for c in range(cdim // 16):
                sl = pl.ds(c * 16, 16)
                acc = gbuf[0, r, sl]
                for k in range(1, KNBRS):
                    acc = jnp.maximum(acc, gbuf[k, r, sl])
                obuf[r, sl] = acc
            return 0

        lax.fori_loop(0, SUBCHUNK, row_max, 0)
        pltpu.sync_copy(obuf,
                        out_ref.at[pl.ds(base + j * SUBCHUNK, SUBCHUNK)])


def _block_diag_weights(w):
    cout = w.shape[0]
    half = w.shape[1]
    wg = w[:, :, 0, 0].reshape(NGROUPS, cout // NGROUPS, half)
    wx = wg[:, :, 0::2]
    wj = wg[:, :, 1::2]
    cin = NGROUPS * (half // 2)
    bx = jnp.zeros((cin, cout), jnp.float32)
    bj = jnp.zeros((cin, cout), jnp.float32)
    go = cout // NGROUPS
    gi = half // 2
    for g in range(NGROUPS):
        bx = bx.at[g * gi:(g + 1) * gi, g * go:(g + 1) * go].set(wx[g].T)
        bj = bj.at[g * gi:(g + 1) * gi, g * go:(g + 1) * go].set(wj[g].T)
    return bx, bj


def kernel(x, w, b):
    B, C, H, W = x.shape
    N = H * W
    cout = w.shape[0]
    xtt = x.reshape(B, C, N)
    xt = jnp.transpose(xtt, (0, 2, 1))

    nblk = N // ROWBLK
    nn_idx = pl.pallas_call(
        _knn_body,
        grid=(B, nblk),
        in_specs=[
            pl.BlockSpec((1, ROWBLK, C), lambda bi, ri: (bi, ri, 0)),
            pl.BlockSpec((1, C, N), lambda bi, ri: (bi, 0, 0)),
        ],
        out_specs=pl.BlockSpec((1, ROWBLK, 16), lambda bi, ri: (bi, ri, 0)),
        out_shape=jax.ShapeDtypeStruct((B, N, 16), jnp.int32),
    )(xt, xtt)

    per_w = (B * N) // NWORKERS
    chunks_per_b = N // per_w
    gidx = jnp.transpose(nn_idx[:, :, :KNBRS], (0, 2, 1))
    gidx = gidx + (jnp.arange(B, dtype=jnp.int32) * N)[:, None, None]
    gidx = gidx.reshape(B, KNBRS, chunks_per_b, per_w)
    gidx = jnp.transpose(gidx, (0, 2, 1, 3)).reshape(-1)

    cpad = 128
    xt_flat = xt.reshape(B * N, C)
    xt_pad = jnp.concatenate(
        [xt_flat, jnp.ones((B * N, 1), jnp.float32),
         jnp.zeros((B * N, cpad - C - 1), jnp.float32)], axis=1)
    mesh = plsc.VectorSubcoreMesh(core_axis_name="c", subcore_axis_name="s")
    maxg = pl.kernel(
        _gather_body,
        mesh=mesh,
        out_type=jax.ShapeDtypeStruct((B * N, cpad), jnp.float32),
        scratch_types=[
            pltpu.VMEM((KNBRS * per_w,), jnp.int32),
            pltpu.VMEM((KNBRS, SUBCHUNK, cpad), jnp.float32),
            pltpu.VMEM((SUBCHUNK, cpad), jnp.float32),
            pltpu.SemaphoreType.DMA,
        ],
    )(xt_pad, gidx)
    maxg = maxg.reshape(B, N, cpad)

    wx, wj = _block_diag_weights(w)
    wxt = (wx - wj).T
    wjt = jnp.concatenate(
        [wj.T, b[:, None], jnp.zeros((cout, cpad - C - 1), jnp.float32)],
        axis=1)
    out = pl.pallas_call(
        _conv_body,
        grid=(B,),
        in_specs=[
            pl.BlockSpec((1, N, C), lambda bi: (bi, 0, 0)),
            pl.BlockSpec((1, N, cpad), lambda bi: (bi, 0, 0)),
            pl.BlockSpec((cout, C), lambda bi: (0, 0)),
            pl.BlockSpec((cout, cpad), lambda bi: (0, 0)),
        ],
        out_specs=pl.BlockSpec((1, cout, N), lambda bi: (bi, 0, 0)),
        out_shape=jax.ShapeDtypeStruct((B, cout, N), jnp.float32),
    )(xt, maxg, wxt, wjt)

    return out.reshape(B, cout, H, W)


# speedup vs baseline: 1.0419x; 1.0419x over previous
import functools

import jax
import jax.numpy as jnp
from jax import lax
from jax.experimental import pallas as pl
from jax.experimental.pallas import tpu as pltpu
from jax.experimental.pallas import tpu_sc as plsc

KNBRS = 9
NGROUPS = 4
ROWBLK = 784
NWORKERS = 32
SUBCHUNK = 48


def _knn_body(xt_ref, xtt_ref, idx_ref):
    xb = xt_ref[0]
    xt_all = xtt_ref[0]
    R = xb.shape[0]
    N = xt_all.shape[1]
    rn = jnp.sqrt(jnp.sum(xb * xb, axis=1, keepdims=True))
    xbn = xb / jnp.maximum(rn, 1e-12)
    cn = jnp.sqrt(jnp.sum(xt_all * xt_all, axis=0, keepdims=True))
    xcn = xt_all / jnp.maximum(cn, 1e-12)
    inner = jnp.dot(xbn, xcn, preferred_element_type=jnp.float32)
    sq_row = jnp.sum(xbn * xbn, axis=1, keepdims=True)
    sq_col = jnp.sum(xcn * xcn, axis=0, keepdims=True)
    neg = 2.0 * inner - sq_row - sq_col
    iota = lax.broadcasted_iota(jnp.int32, (R, N), 1)
    masked_out = jnp.float32(-3.0e38)
    picks = []
    for _ in range(KNBRS):
        a = jnp.argmax(neg, axis=1).astype(jnp.int32)[:, None]
        picks.append(a)
        neg = jnp.where(iota == a, masked_out, neg)
    picks.append(jnp.zeros((R, 16 - len(picks)), jnp.int32))
    idx_ref[0] = jnp.concatenate(picks, axis=1)


def _conv_body(xt_ref, xg_ref, wx_ref, wj_ref, out_ref):
    acc = jnp.dot(xt_ref[0], wx_ref[...], preferred_element_type=jnp.float32)
    acc += jnp.dot(xg_ref[0], wj_ref[...], preferred_element_type=jnp.float32)
    out_ref[0] = jnp.maximum(acc, 0.0)


def _gather_body(xt_ref, gidx_ref, out_ref, idxb, gbuf,
                 isem0, isem1, gsem0, gsem1, osem0, osem1):
    n_total = xt_ref.shape[0]
    cdim = xt_ref.shape[1]
    per_w = n_total // NWORKERS
    wid = lax.axis_index("s") * 2 + lax.axis_index("c")
    base = wid * per_w
    ibase = wid * (KNBRS * per_w)
    isems = (isem0, isem1)
    gsems = (gsem0, gsem1)
    osems = (osem0, osem1)
    nsub = per_w // SUBCHUNK

    tail = per_w - nsub * SUBCHUNK

    def fire_idx(j, s, rows=SUBCHUNK):
        return [
            pltpu.async_copy(
                gidx_ref.at[pl.ds(ibase + k * per_w + j * SUBCHUNK, rows)],
                idxb.at[s, k, pl.ds(0, rows)], isems[s])
            for k in range(KNBRS)
        ]

    def fire_gather(s, rows=SUBCHUNK):
        return [
            pltpu.async_copy(xt_ref.at[idxb.at[s, k, pl.ds(0, rows)]],
                             gbuf.at[s, k, pl.ds(0, rows)], gsems[s])
            for k in range(KNBRS)
        ]

    def reduce_rows(s, rows):
        def row_max(r, _):
            for c in range(cdim // 16):
                sl = pl.ds(c * 16, 16)
                acc = gbuf[s, 0, r, sl]
                for k in range(1, KNBRS):
                    acc = jnp.maximum(acc, gbuf[s, k, r, sl])
                gbuf[s, 0, r, sl] = acc
            return 0

        lax.fori_loop(0, rows, row_max, 0)

    icps = [None] * nsub
    gcps = [None] * nsub
    ocps = [None] * nsub
    icps[0] = fire_idx(0, 0)
    for cp in icps[0]:
        cp.wait()
    gcps[0] = fire_gather(0)
    if nsub > 1:
        icps[1] = fire_idx(1, 1)
    for j in range(nsub):
        s = j % 2
        for cp in gcps[j]:
            cp.wait()
        if j + 1 < nsub:
            for cp in icps[j + 1]:
                cp.wait()
            if j >= 1:
                ocps[j - 1].wait()
            gcps[j + 1] = fire_gather(1 - s)
            if j + 2 < nsub:
                icps[j + 2] = fire_idx(j + 2, s)
        reduce_rows(s, SUBCHUNK)
        ocps[j] = pltpu.async_copy(
            gbuf.at[s, 0, pl.ds(0, SUBCHUNK)],
            out_ref.at[pl.ds(base + j * SUBCHUNK, SUBCHUNK)],
            osems[s])
    if nsub > 1:
        ocps[nsub - 2].wait()
    ocps[nsub - 1].wait()
    if tail:
        tcps = fire_idx(nsub, 0, rows=tail)
        for cp in tcps:
            cp.wait()
        tgcps = fire_gather(0, rows=tail)
        for cp in tgcps:
            cp.wait()
        reduce_rows(0, tail)
        pltpu.async_copy(
            gbuf.at[0, 0, pl.ds(0, tail)],
            out_ref.at[pl.ds(base + nsub * SUBCHUNK, tail)],
            osems[0]).wait()


def _block_diag_weights(w):
    cout = w.shape[0]
    half = w.shape[1]
    wg = w[:, :, 0, 0].reshape(NGROUPS, cout // NGROUPS, half)
    wx = wg[:, :, 0::2]
    wj = wg[:, :, 1::2]
    cin = NGROUPS * (half // 2)
    bx = jnp.zeros((cin, cout), jnp.float32)
    bj = jnp.zeros((cin, cout), jnp.float32)
    go = cout // NGROUPS
    gi = half // 2
    for g in range(NGROUPS):
        bx = bx.at[g * gi:(g + 1) * gi, g * go:(g + 1) * go].set(wx[g].T)
        bj = bj.at[g * gi:(g + 1) * gi, g * go:(g + 1) * go].set(wj[g].T)
    return bx, bj


def kernel(x, w, b):
    B, C, H, W = x.shape
    N = H * W
    cout = w.shape[0]
    xtt = x.reshape(B, C, N)
    xt = jnp.transpose(xtt, (0, 2, 1))

    nblk = N // ROWBLK
    nn_idx = pl.pallas_call(
        _knn_body,
        grid=(B, nblk),
        in_specs=[
            pl.BlockSpec((1, ROWBLK, C), lambda bi, ri: (bi, ri, 0)),
            pl.BlockSpec((1, C, N), lambda bi, ri: (bi, 0, 0)),
        ],
        out_specs=pl.BlockSpec((1, ROWBLK, 16), lambda bi, ri: (bi, ri, 0)),
        out_shape=jax.ShapeDtypeStruct((B, N, 16), jnp.int32),
    )(xt, xtt)

    per_w = (B * N) // NWORKERS
    chunks_per_b = N // per_w
    gidx = jnp.transpose(nn_idx[:, :, :KNBRS], (0, 2, 1))
    gidx = gidx + (jnp.arange(B, dtype=jnp.int32) * N)[:, None, None]
    gidx = gidx.reshape(B, KNBRS, chunks_per_b, per_w)
    gidx = jnp.transpose(gidx, (0, 2, 1, 3)).reshape(-1)

    cpad = 128
    xt_flat = xt.reshape(B * N, C)
    xt_pad = jnp.concatenate(
        [xt_flat, jnp.ones((B * N, 1), jnp.float32),
         jnp.zeros((B * N, cpad - C - 1), jnp.float32)], axis=1)
    mesh = plsc.VectorSubcoreMesh(core_axis_name="c", subcore_axis_name="s")
    maxg = pl.kernel(
        _gather_body,
        mesh=mesh,
        out_type=jax.ShapeDtypeStruct((B * N, cpad), jnp.float32),
        scratch_types=[
            pltpu.VMEM((2, KNBRS, SUBCHUNK), jnp.int32),
            pltpu.VMEM((2, KNBRS, SUBCHUNK, cpad), jnp.float32),
            pltpu.SemaphoreType.DMA,
            pltpu.SemaphoreType.DMA,
            pltpu.SemaphoreType.DMA,
            pltpu.SemaphoreType.DMA,
            pltpu.SemaphoreType.DMA,
            pltpu.SemaphoreType.DMA,
        ],
    )(xt_pad, gidx)
    maxg = maxg.reshape(B, N, cpad)

    wx, wj = _block_diag_weights(w)
    wxm = wx - wj
    wjp = jnp.concatenate(
        [wj, b[None, :], jnp.zeros((cpad - C - 1, cout), jnp.float32)],
        axis=0)
    out = pl.pallas_call(
        _conv_body,
        grid=(B, nblk),
        in_specs=[
            pl.BlockSpec((1, ROWBLK, C), lambda bi, ri: (bi, ri, 0)),
            pl.BlockSpec((1, ROWBLK, cpad), lambda bi, ri: (bi, ri, 0)),
            pl.BlockSpec((C, cout), lambda bi, ri: (0, 0)),
            pl.BlockSpec((cpad, cout), lambda bi, ri: (0, 0)),
        ],
        out_specs=pl.BlockSpec((1, ROWBLK, cout), lambda bi, ri: (bi, ri, 0)),
        out_shape=jax.ShapeDtypeStruct((B, N, cout), jnp.float32),
    )(xt, maxg, wxm, wjp)

    return jnp.transpose(out, (0, 2, 1)).reshape(B, cout, H, W)
